# Initial kernel scaffold; baseline (speedup 1.0000x reference)
import functools

import jax
import jax.numpy as jnp
from jax import lax
from jax.experimental import pallas as pl
from jax.experimental.pallas import tpu as pltpu
from jax.experimental.pallas import tpu_sc as plsc

_NC = 2
_NS = 16
_ROWS = 400


def _segment_sum_sc(h, src, dst):
    N, D = h.shape
    E = src.shape[0]
    NW = _NC * _NS
    epw = E // NW
    CHUNK = 80
    n_chunks = epw // CHUNK
    assert epw * NW == E and n_chunks * CHUNK == epw
    assert N % _NS == 0 and D % 16 == 0
    rows_per_tile = N // _NS
    n_full = rows_per_tile // CHUNK
    rem = rows_per_tile - n_full * CHUNK
    mesh = plsc.VectorSubcoreMesh(core_axis_name="c", subcore_axis_name="s")

    @functools.partial(
        pl.kernel,
        out_type=jax.ShapeDtypeStruct((_NC, N, D), jnp.float32),
        mesh=mesh,
        scratch_types=[
            pltpu.VMEM((CHUNK,), jnp.int32),
            pltpu.VMEM((CHUNK,), jnp.int32),
            pltpu.VMEM((CHUNK, D), jnp.float32),
            pltpu.VMEM_SHARED((N, D), jnp.float32),
        ],
    )
    def k(h_hbm, src_hbm, dst_hbm, out_hbm, src_v, dst_v, rows_v, agg_sh):
        cid = lax.axis_index("c")
        sid = lax.axis_index("s")
        wid = cid * _NS + sid

        @pl.loop(0, CHUNK)
        def _(i):
            @pl.loop(0, D, step=16)
            def _(j):
                rows_v[i, pl.ds(j, 16)] = jnp.zeros((16,), jnp.float32)

        base_r = sid * rows_per_tile

        @pl.loop(0, n_full)
        def _(t):
            pltpu.sync_copy(rows_v, agg_sh.at[pl.ds(base_r + t * CHUNK, CHUNK)])

        if rem:
            pltpu.sync_copy(rows_v.at[pl.ds(0, rem)],
                            agg_sh.at[pl.ds(base_r + n_full * CHUNK, rem)])
        plsc.subcore_barrier()

        e_base = wid * epw

        @pl.loop(0, n_chunks)
        def _(t):
            e0 = e_base + t * CHUNK
            pltpu.sync_copy(src_hbm.at[pl.ds(e0, CHUNK)], src_v)
            pltpu.sync_copy(dst_hbm.at[pl.ds(e0, CHUNK)], dst_v)
            pltpu.sync_copy(h_hbm.at[src_v], rows_v)
            pltpu.sync_copy(rows_v, agg_sh.at[dst_v], add=True)

        plsc.subcore_barrier()
        pltpu.sync_copy(agg_sh.at[pl.ds(base_r, rows_per_tile)],
                        out_hbm.at[cid, pl.ds(base_r, rows_per_tile)])

    return k(h, src, dst)


def _concat_tc(x, disease_vec, batch_col):
    N, DF = x.shape
    Bg, DD = disease_vec.shape
    G = N // _ROWS

    def body(x_ref, dis_ref, b_ref, o_ref):
        iota = lax.broadcasted_iota(jnp.float32, (_ROWS, Bg), 1)
        oh = (b_ref[...] == iota).astype(jnp.float32)
        d = jnp.dot(oh, dis_ref[...], preferred_element_type=jnp.float32)
        o_ref[:, :DF] = x_ref[...]
        o_ref[:, DF:] = d

    return pl.pallas_call(
        body,
        grid=(G,),
        in_specs=[
            pl.BlockSpec((_ROWS, DF), lambda i: (i, 0)),
            pl.BlockSpec((Bg, DD), lambda i: (0, 0)),
            pl.BlockSpec((_ROWS, 1), lambda i: (i, 0)),
        ],
        out_specs=pl.BlockSpec((_ROWS, DF + DD), lambda i: (i, 0)),
        out_shape=jax.ShapeDtypeStruct((N, DF + DD), jnp.float32),
    )(x, disease_vec, batch_col)


def _gin_mlp_tc(h, agg, scale, W1, b1, W2, b2):
    N, Din = h.shape
    Dh = W1.shape[1]
    G = N // _ROWS

    def body(h_ref, a0_ref, a1_ref, s_ref, W1_ref, b1_ref, W2_ref, b2_ref,
             o_ref):
        z = h_ref[...] * s_ref[...] + a0_ref[0] + a1_ref[0]
        z = jnp.dot(z, W1_ref[...], preferred_element_type=jnp.float32)
        z = jnp.maximum(z + b1_ref[...], 0.0)
        z = jnp.dot(z, W2_ref[...], preferred_element_type=jnp.float32)
        o_ref[...] = jnp.maximum(z + b2_ref[...], 0.0)

    return pl.pallas_call(
        body,
        grid=(G,),
        in_specs=[
            pl.BlockSpec((_ROWS, Din), lambda i: (i, 0)),
            pl.BlockSpec((1, _ROWS, Din), lambda i: (0, i, 0)),
            pl.BlockSpec((1, _ROWS, Din), lambda i: (1, i, 0)),
            pl.BlockSpec((1, 1), lambda i: (0, 0)),
            pl.BlockSpec((Din, Dh), lambda i: (0, 0)),
            pl.BlockSpec((1, Dh), lambda i: (0, 0)),
            pl.BlockSpec((Dh, Dh), lambda i: (0, 0)),
            pl.BlockSpec((1, Dh), lambda i: (0, 0)),
        ],
        out_specs=pl.BlockSpec((_ROWS, Dh), lambda i: (i, 0)),
        out_shape=jax.ShapeDtypeStruct((N, Dh), jnp.float32),
    )(h, agg, agg, scale, W1, b1, W2, b2)


def _gin_final_tc(h, agg, scale, W1, b1, W2, b2, batch_row,
                  W_mu, b_mu, W_lv, b_lv):
    N, Din = h.shape
    Dh = W1.shape[1]
    LAT = W_mu.shape[1]
    G, _ = batch_row.shape
    Bg = W_mu.shape[0] and b_mu.shape[1]
    Bg = 64

    def body(h_ref, a0_ref, a1_ref, s_ref, W1_ref, b1_ref, W2_ref, b2_ref,
             b_row_ref, Wmu_ref, bmu_ref, Wlv_ref, blv_ref,
             mu_ref, lv_ref, g_acc):
        i = pl.program_id(0)
        z = h_ref[...] * s_ref[...] + a0_ref[0] + a1_ref[0]
        z = jnp.dot(z, W1_ref[...], preferred_element_type=jnp.float32)
        z = jnp.maximum(z + b1_ref[...], 0.0)
        z = jnp.dot(z, W2_ref[...], preferred_element_type=jnp.float32)
        z = jnp.maximum(z + b2_ref[...], 0.0)
        iota = lax.broadcasted_iota(jnp.float32, (Bg, _ROWS), 0)
        ohT = (b_row_ref[...] == iota).astype(jnp.float32)
        g = jnp.dot(ohT, z, preferred_element_type=jnp.float32)

        @pl.when(i == 0)
        def _():
            g_acc[...] = g

        @pl.when(i > 0)
        def _():
            g_acc[...] = g_acc[...] + g

        @pl.when(i == G - 1)
        def _():
            gg = g_acc[...]
            mu_ref[...] = (jnp.dot(gg, Wmu_ref[...],
                                   preferred_element_type=jnp.float32)
                           + bmu_ref[...])
            lv_ref[...] = (jnp.dot(gg, Wlv_ref[...],
                                   preferred_element_type=jnp.float32)
                           + blv_ref[...])

    return pl.pallas_call(
        body,
        grid=(G,),
        in_specs=[
            pl.BlockSpec((_ROWS, Din), lambda i: (i, 0)),
            pl.BlockSpec((1, _ROWS, Din), lambda i: (0, i, 0)),
            pl.BlockSpec((1, _ROWS, Din), lambda i: (1, i, 0)),
            pl.BlockSpec((1, 1), lambda i: (0, 0)),
            pl.BlockSpec((Din, Dh), lambda i: (0, 0)),
            pl.BlockSpec((1, Dh), lambda i: (0, 0)),
            pl.BlockSpec((Dh, Dh), lambda i: (0, 0)),
            pl.BlockSpec((1, Dh), lambda i: (0, 0)),
            pl.BlockSpec((1, _ROWS), lambda i: (i, 0)),
            pl.BlockSpec((Dh, LAT), lambda i: (0, 0)),
            pl.BlockSpec((1, LAT), lambda i: (0, 0)),
            pl.BlockSpec((Dh, LAT), lambda i: (0, 0)),
            pl.BlockSpec((1, LAT), lambda i: (0, 0)),
        ],
        out_specs=[
            pl.BlockSpec((Bg, LAT), lambda i: (0, 0)),
            pl.BlockSpec((Bg, LAT), lambda i: (0, 0)),
        ],
        out_shape=[
            jax.ShapeDtypeStruct((Bg, LAT), jnp.float32),
            jax.ShapeDtypeStruct((Bg, LAT), jnp.float32),
        ],
        scratch_shapes=[pltpu.VMEM((Bg, Dh), jnp.float32)],
    )(h, agg, agg, scale, W1, b1, W2, b2, batch_row,
      W_mu, b_mu, W_lv, b_lv)


def kernel(x, edge_index, batch, disease_vec,
           W1_0, b1_0, W2_0, b2_0, eps_0,
           W1_1, b1_1, W2_1, b2_1, eps_1,
           W1_2, b1_2, W2_2, b2_2, eps_2,
           W_mu, b_mu, W_lv, b_lv):
    N = x.shape[0]
    src = edge_index[0]
    dst = edge_index[1]
    batch_f = batch.astype(jnp.float32)
    batch_col = batch_f.reshape(N, 1)
    G = N // _ROWS
    batch_row = batch_f.reshape(G, _ROWS)

    h = _concat_tc(x, disease_vec, batch_col)

    layers = [
        (W1_0, b1_0.reshape(1, -1), W2_0, b2_0.reshape(1, -1), eps_0),
        (W1_1, b1_1.reshape(1, -1), W2_1, b2_1.reshape(1, -1), eps_1),
        (W1_2, b1_2.reshape(1, -1), W2_2, b2_2.reshape(1, -1), eps_2),
    ]
    for li, (W1, b1, W2, b2, eps) in enumerate(layers):
        agg = _segment_sum_sc(h, src, dst)
        scale = (1.0 + eps).reshape(1, 1)
        if li < 2:
            h = _gin_mlp_tc(h, agg, scale, W1, b1, W2, b2)
        else:
            mu, lv = _gin_final_tc(
                h, agg, scale, W1, b1, W2, b2, batch_row,
                W_mu, b_mu.reshape(1, -1), W_lv, b_lv.reshape(1, -1))
    return (mu, lv)


# speedup vs baseline: 5.7964x; 5.7964x over previous
import functools

import jax
import jax.numpy as jnp
from jax import lax
from jax.experimental import pallas as pl
from jax.experimental.pallas import tpu as pltpu
from jax.experimental.pallas import tpu_sc as plsc

_NC = 2
_NS = 16
_ROWS = 400


def _segment_sum_sc(h, src, dst):
    N, D = h.shape
    E = src.shape[0]
    NW = _NC * _NS
    CHUNK = 128
    RCHUNK = 80
    n_ec = E // CHUNK
    n_rc = N // RCHUNK
    assert n_ec * CHUNK == E and n_rc * RCHUNK == N and D % 16 == 0
    ec_full, ec_rem = divmod(n_ec, NW)
    rc_full, rc_rem = divmod(n_rc, _NS)
    mesh = plsc.VectorSubcoreMesh(core_axis_name="c", subcore_axis_name="s")

    @functools.partial(
        pl.kernel,
        out_type=jax.ShapeDtypeStruct((_NC, N, D), jnp.float32),
        mesh=mesh,
        scratch_types=[
            pltpu.VMEM((CHUNK,), jnp.int32),
            pltpu.VMEM((CHUNK,), jnp.int32),
            pltpu.VMEM((CHUNK, D), jnp.float32),
            pltpu.VMEM_SHARED((N, D), jnp.float32),
        ],
    )
    def k(h_hbm, src_hbm, dst_hbm, out_hbm, src_v, dst_v, rows_v, agg_sh):
        cid = lax.axis_index("c")
        sid = lax.axis_index("s")
        wid = cid * _NS + sid

        @pl.loop(0, RCHUNK)
        def _(i):
            @pl.loop(0, D, step=16)
            def _(j):
                rows_v[i, pl.ds(j, 16)] = jnp.zeros((16,), jnp.float32)

        def zero_rows(c):
            pltpu.sync_copy(rows_v.at[pl.ds(0, RCHUNK)],
                            agg_sh.at[pl.ds(c * RCHUNK, RCHUNK)])

        @pl.loop(0, rc_full)
        def _(t):
            zero_rows(t * _NS + sid)

        if rc_rem:
            @pl.when(sid < rc_rem)
            def _():
                zero_rows(rc_full * _NS + sid)

        plsc.subcore_barrier()

        def do_chunk(c):
            e0 = c * CHUNK
            pltpu.sync_copy(src_hbm.at[pl.ds(e0, CHUNK)], src_v)
            pltpu.sync_copy(dst_hbm.at[pl.ds(e0, CHUNK)], dst_v)
            pltpu.sync_copy(h_hbm.at[src_v], rows_v)
            pltpu.sync_copy(rows_v, agg_sh.at[dst_v], add=True)

        @pl.loop(0, ec_full)
        def _(t):
            do_chunk(t * NW + wid)

        if ec_rem:
            @pl.when(wid < ec_rem)
            def _():
                do_chunk(ec_full * NW + wid)

        plsc.subcore_barrier()

        def out_rows(c):
            pltpu.sync_copy(agg_sh.at[pl.ds(c * RCHUNK, RCHUNK)],
                            out_hbm.at[cid, pl.ds(c * RCHUNK, RCHUNK)])

        @pl.loop(0, rc_full)
        def _(t):
            out_rows(t * _NS + sid)

        if rc_rem:
            @pl.when(sid < rc_rem)
            def _():
                out_rows(rc_full * _NS + sid)

    return k(h, src, dst)


def _pre0_tc(x, disease_vec, batch_col, W1_0):
    N, DF = x.shape
    Bg, DD = disease_vec.shape
    Dh = W1_0.shape[1]
    G = N // _ROWS

    def body(x_ref, dis_ref, b_ref, W1_ref, o_ref):
        iota = lax.broadcasted_iota(jnp.int32, (_ROWS, Bg), 1).astype(
            jnp.float32)
        oh = (b_ref[...] == iota).astype(jnp.float32)
        du = jnp.dot(dis_ref[...], W1_ref[DF:, :],
                     preferred_element_type=jnp.float32)
        q = jnp.dot(x_ref[...], W1_ref[:DF, :],
                    preferred_element_type=jnp.float32)
        q = q + jnp.dot(oh, du, preferred_element_type=jnp.float32)
        o_ref[...] = q

    return pl.pallas_call(
        body,
        grid=(G,),
        in_specs=[
            pl.BlockSpec((_ROWS, DF), lambda i: (i, 0)),
            pl.BlockSpec((Bg, DD), lambda i: (0, 0)),
            pl.BlockSpec((_ROWS, 1), lambda i: (i, 0)),
            pl.BlockSpec((DF + DD, Dh), lambda i: (0, 0)),
        ],
        out_specs=pl.BlockSpec((_ROWS, Dh), lambda i: (i, 0)),
        out_shape=jax.ShapeDtypeStruct((N, Dh), jnp.float32),
    )(x, disease_vec, batch_col, W1_0)


def _gin0_mlp_tc(q, agg, scale, b1, W2, b2):
    N, Dh = q.shape
    G = N // _ROWS

    def body(q_ref, a0_ref, a1_ref, s_ref, b1_ref, W2_ref, b2_ref, o_ref):
        z = q_ref[...] * s_ref[...] + a0_ref[0] + a1_ref[0]
        z = jnp.maximum(z + b1_ref[...], 0.0)
        z = jnp.dot(z, W2_ref[...], preferred_element_type=jnp.float32)
        o_ref[...] = jnp.maximum(z + b2_ref[...], 0.0)

    return pl.pallas_call(
        body,
        grid=(G,),
        in_specs=[
            pl.BlockSpec((_ROWS, Dh), lambda i: (i, 0)),
            pl.BlockSpec((1, _ROWS, Dh), lambda i: (0, i, 0)),
            pl.BlockSpec((1, _ROWS, Dh), lambda i: (1, i, 0)),
            pl.BlockSpec((1, 1), lambda i: (0, 0)),
            pl.BlockSpec((1, Dh), lambda i: (0, 0)),
            pl.BlockSpec((Dh, Dh), lambda i: (0, 0)),
            pl.BlockSpec((1, Dh), lambda i: (0, 0)),
        ],
        out_specs=pl.BlockSpec((_ROWS, Dh), lambda i: (i, 0)),
        out_shape=jax.ShapeDtypeStruct((N, Dh), jnp.float32),
    )(q, agg, agg, scale, b1, W2, b2)


def _gin_mlp_tc(h, agg, scale, W1, b1, W2, b2):
    N, Din = h.shape
    Dh = W1.shape[1]
    G = N // _ROWS

    def body(h_ref, a0_ref, a1_ref, s_ref, W1_ref, b1_ref, W2_ref, b2_ref,
             o_ref):
        z = h_ref[...] * s_ref[...] + a0_ref[0] + a1_ref[0]
        z = jnp.dot(z, W1_ref[...], preferred_element_type=jnp.float32)
        z = jnp.maximum(z + b1_ref[...], 0.0)
        z = jnp.dot(z, W2_ref[...], preferred_element_type=jnp.float32)
        o_ref[...] = jnp.maximum(z + b2_ref[...], 0.0)

    return pl.pallas_call(
        body,
        grid=(G,),
        in_specs=[
            pl.BlockSpec((_ROWS, Din), lambda i: (i, 0)),
            pl.BlockSpec((1, _ROWS, Din), lambda i: (0, i, 0)),
            pl.BlockSpec((1, _ROWS, Din), lambda i: (1, i, 0)),
            pl.BlockSpec((1, 1), lambda i: (0, 0)),
            pl.BlockSpec((Din, Dh), lambda i: (0, 0)),
            pl.BlockSpec((1, Dh), lambda i: (0, 0)),
            pl.BlockSpec((Dh, Dh), lambda i: (0, 0)),
            pl.BlockSpec((1, Dh), lambda i: (0, 0)),
        ],
        out_specs=pl.BlockSpec((_ROWS, Dh), lambda i: (i, 0)),
        out_shape=jax.ShapeDtypeStruct((N, Dh), jnp.float32),
    )(h, agg, agg, scale, W1, b1, W2, b2)


def _gin_final_tc(h, agg, scale, W1, b1, W2, b2, batch_row, num_graphs,
                  W_mu, b_mu, W_lv, b_lv):
    N, Din = h.shape
    Dh = W1.shape[1]
    LAT = W_mu.shape[1]
    G = batch_row.shape[0]
    Bg = num_graphs

    def body(h_ref, a0_ref, a1_ref, s_ref, W1_ref, b1_ref, W2_ref, b2_ref,
             b_row_ref, Wmu_ref, bmu_ref, Wlv_ref, blv_ref,
             mu_ref, lv_ref, g_acc):
        i = pl.program_id(0)
        z = h_ref[...] * s_ref[...] + a0_ref[0] + a1_ref[0]
        z = jnp.dot(z, W1_ref[...], preferred_element_type=jnp.float32)
        z = jnp.maximum(z + b1_ref[...], 0.0)
        z = jnp.dot(z, W2_ref[...], preferred_element_type=jnp.float32)
        z = jnp.maximum(z + b2_ref[...], 0.0)
        iota = lax.broadcasted_iota(jnp.int32, (Bg, _ROWS), 0).astype(
            jnp.float32)
        ohT = (b_row_ref[0] == iota).astype(jnp.float32)
        g = jnp.dot(ohT, z, preferred_element_type=jnp.float32)

        @pl.when(i == 0)
        def _():
            g_acc[...] = g

        @pl.when(i > 0)
        def _():
            g_acc[...] = g_acc[...] + g

        @pl.when(i == G - 1)
        def _():
            gg = g_acc[...]
            mu_ref[...] = (jnp.dot(gg, Wmu_ref[...],
                                   preferred_element_type=jnp.float32)
                           + bmu_ref[...])
            lv_ref[...] = (jnp.dot(gg, Wlv_ref[...],
                                   preferred_element_type=jnp.float32)
                           + blv_ref[...])

    return pl.pallas_call(
        body,
        grid=(G,),
        in_specs=[
            pl.BlockSpec((_ROWS, Din), lambda i: (i, 0)),
            pl.BlockSpec((1, _ROWS, Din), lambda i: (0, i, 0)),
            pl.BlockSpec((1, _ROWS, Din), lambda i: (1, i, 0)),
            pl.BlockSpec((1, 1), lambda i: (0, 0)),
            pl.BlockSpec((Din, Dh), lambda i: (0, 0)),
            pl.BlockSpec((1, Dh), lambda i: (0, 0)),
            pl.BlockSpec((Dh, Dh), lambda i: (0, 0)),
            pl.BlockSpec((1, Dh), lambda i: (0, 0)),
            pl.BlockSpec((1, 1, _ROWS), lambda i: (i, 0, 0)),
            pl.BlockSpec((Dh, LAT), lambda i: (0, 0)),
            pl.BlockSpec((1, LAT), lambda i: (0, 0)),
            pl.BlockSpec((Dh, LAT), lambda i: (0, 0)),
            pl.BlockSpec((1, LAT), lambda i: (0, 0)),
        ],
        out_specs=[
            pl.BlockSpec((Bg, LAT), lambda i: (0, 0)),
            pl.BlockSpec((Bg, LAT), lambda i: (0, 0)),
        ],
        out_shape=[
            jax.ShapeDtypeStruct((Bg, LAT), jnp.float32),
            jax.ShapeDtypeStruct((Bg, LAT), jnp.float32),
        ],
        scratch_shapes=[pltpu.VMEM((Bg, Dh), jnp.float32)],
    )(h, agg, agg, scale, W1, b1, W2, b2, batch_row,
      W_mu, b_mu, W_lv, b_lv)


def kernel(x, edge_index, batch, disease_vec,
           W1_0, b1_0, W2_0, b2_0, eps_0,
           W1_1, b1_1, W2_1, b2_1, eps_1,
           W1_2, b1_2, W2_2, b2_2, eps_2,
           W_mu, b_mu, W_lv, b_lv):
    N = x.shape[0]
    src = edge_index[0]
    dst = edge_index[1]
    batch_f = batch.astype(jnp.float32)
    batch_col = batch_f.reshape(N, 1)
    G = N // _ROWS
    batch_row = batch_f.reshape(G, 1, _ROWS)

    q = _pre0_tc(x, disease_vec, batch_col, W1_0)
    agg = _segment_sum_sc(q, src, dst)
    h = _gin0_mlp_tc(q, agg, (1.0 + eps_0).reshape(1, 1),
                     b1_0.reshape(1, -1), W2_0, b2_0.reshape(1, -1))

    agg = _segment_sum_sc(h, src, dst)
    h = _gin_mlp_tc(h, agg, (1.0 + eps_1).reshape(1, 1),
                    W1_1, b1_1.reshape(1, -1), W2_1, b2_1.reshape(1, -1))

    agg = _segment_sum_sc(h, src, dst)
    mu, lv = _gin_final_tc(
        h, agg, (1.0 + eps_2).reshape(1, 1), W1_2, b1_2.reshape(1, -1),
        W2_2, b2_2.reshape(1, -1), batch_row, disease_vec.shape[0],
        W_mu, b_mu.reshape(1, -1), W_lv, b_lv.reshape(1, -1))
    return (mu, lv)
